# baseline (speedup 1.0000x reference)
import jax
import jax.numpy as jnp
from jax.experimental import pallas as pl
from jax.experimental.pallas import tpu as pltpu

_N, _D, _S = 100000, 128, 512
_RED = _D // 4
_B = 10000
_NB = _N // _B


def _onehot(b_ref):
    bb = b_ref[0]
    seg_iota = jax.lax.broadcasted_iota(jnp.int32, (_S, _B), 0)
    return (seg_iota == bb).astype(jnp.float32)


def _pass1_body(x_ref, b_ref, f1w_ref, f1b_ref, f2w_ref, f2b_ref,
                x2_ref, sums_ref, counts_ref):
    i = pl.program_id(0)

    @pl.when(i == 0)
    def _init():
        sums_ref[...] = jnp.zeros_like(sums_ref)
        counts_ref[...] = jnp.zeros_like(counts_ref)

    x = x_ref[...]
    h = jnp.maximum(
        jax.lax.dot_general(x, f1w_ref[...], (((1,), (1,)), ((), ())),
                            preferred_element_type=jnp.float32) + f1b_ref[...],
        0.0)
    att = jnp.tanh(
        jax.lax.dot_general(h, f2w_ref[...], (((1,), (1,)), ((), ())),
                            preferred_element_type=jnp.float32) + f2b_ref[...])
    x2 = att * x + x
    x2_ref[...] = x2.astype(jnp.bfloat16)

    oh = _onehot(b_ref)
    sums_ref[...] += jax.lax.dot(oh, x2, preferred_element_type=jnp.float32)
    counts_ref[...] += jnp.sum(oh, axis=1, keepdims=True)


def _pass2_body(x2_ref, b_ref, w_ref, sums_ref, counts_ref, out_ref, tg):
    i = pl.program_id(0)

    @pl.when(i == 0)
    def _mid():
        mean = sums_ref[...] / jnp.maximum(counts_ref[...], 1.0)
        tg[...] = jnp.tanh(jax.lax.dot(
            mean, w_ref[...], preferred_element_type=jnp.float32))
        out_ref[...] = jnp.zeros_like(out_ref)

    x2 = x2_ref[...].astype(jnp.float32)
    oh = _onehot(b_ref)
    trT = jax.lax.dot_general(tg[...], oh, (((0,), (0,)), ((), ())),
                              preferred_element_type=jnp.float32)
    tg_rows = trT.T
    ones_col = jnp.ones((_D, 8), jnp.float32)
    pre = jax.lax.dot_general(x2 * tg_rows, ones_col, (((1,), (0,)), ((), ())),
                              preferred_element_type=jnp.float32)
    coefs = jax.nn.sigmoid(pre[:, 0:1])
    out_ref[...] += jax.lax.dot(oh, coefs * x2,
                                preferred_element_type=jnp.float32)


def kernel(x, batch, size, weight_matrix, fc1_w, fc1_b, fc2_w, fc2_b):
    batch = batch.astype(jnp.int32)
    offset = jnp.asarray(size, jnp.int32) - jnp.int32(_S)
    batch = (batch + offset).reshape(_NB, 1, _B)
    fc1_b2 = fc1_b.reshape(1, _RED)
    fc2_b2 = fc2_b.reshape(1, _D)

    x2_bf, sums, counts = pl.pallas_call(
        _pass1_body,
        grid=(_NB,),
        in_specs=[
            pl.BlockSpec((_B, _D), lambda i: (i, 0)),
            pl.BlockSpec((1, 1, _B), lambda i: (i, 0, 0)),
            pl.BlockSpec((_RED, _D), lambda i: (0, 0)),
            pl.BlockSpec((1, _RED), lambda i: (0, 0)),
            pl.BlockSpec((_D, _RED), lambda i: (0, 0)),
            pl.BlockSpec((1, _D), lambda i: (0, 0)),
        ],
        out_specs=[
            pl.BlockSpec((_B, _D), lambda i: (i, 0)),
            pl.BlockSpec((_S, _D), lambda i: (0, 0)),
            pl.BlockSpec((_S, 1), lambda i: (0, 0)),
        ],
        out_shape=[
            jax.ShapeDtypeStruct((_N, _D), jnp.bfloat16),
            jax.ShapeDtypeStruct((_S, _D), jnp.float32),
            jax.ShapeDtypeStruct((_S, 1), jnp.float32),
        ],
    )(x, batch, fc1_w, fc1_b2, fc2_w, fc2_b2)

    return pl.pallas_call(
        _pass2_body,
        grid=(_NB,),
        in_specs=[
            pl.BlockSpec((_B, _D), lambda i: (i, 0)),
            pl.BlockSpec((1, 1, _B), lambda i: (i, 0, 0)),
            pl.BlockSpec((_D, _D), lambda i: (0, 0)),
            pl.BlockSpec((_S, _D), lambda i: (0, 0)),
            pl.BlockSpec((_S, 1), lambda i: (0, 0)),
        ],
        out_specs=pl.BlockSpec((_S, _D), lambda i: (0, 0)),
        out_shape=jax.ShapeDtypeStruct((_S, _D), jnp.float32),
        scratch_shapes=[
            pltpu.VMEM((_S, _D), jnp.float32),
        ],
    )(x2_bf, batch, weight_matrix, sums, counts)


# speedup vs baseline: 1.1762x; 1.1762x over previous
import jax
import jax.numpy as jnp
from jax.experimental import pallas as pl
from jax.experimental.pallas import tpu as pltpu

_N, _D, _S = 100000, 128, 512
_RED = _D // 4
_B = 10000
_NB = _N // _B


def _onehot(b_ref):
    bb = b_ref[0]
    seg_iota = jax.lax.broadcasted_iota(jnp.int32, (_S, _B), 0)
    return (seg_iota == bb).astype(jnp.float32)


def _pass1_body(x_ref, b_ref, f1w_ref, f1b_ref, f2w_ref, f2b_ref,
                x2_ref, sums_ref, counts_ref):
    i = pl.program_id(0)

    @pl.when(i == 0)
    def _init():
        sums_ref[...] = jnp.zeros_like(sums_ref)
        counts_ref[...] = jnp.zeros_like(counts_ref)

    x = x_ref[...]
    h = jnp.maximum(
        jax.lax.dot_general(x, f1w_ref[...], (((1,), (1,)), ((), ())),
                            preferred_element_type=jnp.float32) + f1b_ref[...],
        0.0)
    att = jnp.tanh(
        jax.lax.dot_general(h, f2w_ref[...], (((1,), (1,)), ((), ())),
                            preferred_element_type=jnp.float32) + f2b_ref[...])
    x2 = att * x + x
    x2_ref[...] = x2.astype(jnp.bfloat16)

    oh = _onehot(b_ref)
    rhs = jnp.concatenate([x2, jnp.ones((_B, _D), jnp.float32)], axis=1)
    se = jax.lax.dot(oh, rhs, preferred_element_type=jnp.float32)
    sums_ref[...] += se[:, :_D]
    counts_ref[...] += se[:, _D:_D + 1]


def _pass2_body(x2_ref, b_ref, w_ref, sums_ref, counts_ref, out_ref, tg):
    i = pl.program_id(0)

    @pl.when(i == 0)
    def _mid():
        mean = sums_ref[...] / jnp.maximum(counts_ref[...], 1.0)
        tg[...] = jnp.tanh(jax.lax.dot(
            mean, w_ref[...], preferred_element_type=jnp.float32))
        out_ref[...] = jnp.zeros_like(out_ref)

    x2 = x2_ref[...].astype(jnp.float32)
    oh = _onehot(b_ref)
    trT = jax.lax.dot_general(tg[...], oh, (((0,), (0,)), ((), ())),
                              preferred_element_type=jnp.float32)
    tg_rows = trT.T
    coefs = jax.nn.sigmoid(jnp.sum(x2 * tg_rows, axis=1, keepdims=True))
    out_ref[...] += jax.lax.dot(oh, coefs * x2,
                                preferred_element_type=jnp.float32)


def kernel(x, batch, size, weight_matrix, fc1_w, fc1_b, fc2_w, fc2_b):
    batch = batch.astype(jnp.int32)
    offset = jnp.asarray(size, jnp.int32) - jnp.int32(_S)
    batch = (batch + offset).reshape(_NB, 1, _B)
    fc1_b2 = fc1_b.reshape(1, _RED)
    fc2_b2 = fc2_b.reshape(1, _D)

    x2_bf, sums, counts = pl.pallas_call(
        _pass1_body,
        grid=(_NB,),
        in_specs=[
            pl.BlockSpec((_B, _D), lambda i: (i, 0)),
            pl.BlockSpec((1, 1, _B), lambda i: (i, 0, 0)),
            pl.BlockSpec((_RED, _D), lambda i: (0, 0)),
            pl.BlockSpec((1, _RED), lambda i: (0, 0)),
            pl.BlockSpec((_D, _RED), lambda i: (0, 0)),
            pl.BlockSpec((1, _D), lambda i: (0, 0)),
        ],
        out_specs=[
            pl.BlockSpec((_B, _D), lambda i: (i, 0)),
            pl.BlockSpec((_S, _D), lambda i: (0, 0)),
            pl.BlockSpec((_S, 1), lambda i: (0, 0)),
        ],
        out_shape=[
            jax.ShapeDtypeStruct((_N, _D), jnp.bfloat16),
            jax.ShapeDtypeStruct((_S, _D), jnp.float32),
            jax.ShapeDtypeStruct((_S, 1), jnp.float32),
        ],
    )(x, batch, fc1_w, fc1_b2, fc2_w, fc2_b2)

    return pl.pallas_call(
        _pass2_body,
        grid=(_NB,),
        in_specs=[
            pl.BlockSpec((_B, _D), lambda i: (i, 0)),
            pl.BlockSpec((1, 1, _B), lambda i: (i, 0, 0)),
            pl.BlockSpec((_D, _D), lambda i: (0, 0)),
            pl.BlockSpec((_S, _D), lambda i: (0, 0)),
            pl.BlockSpec((_S, 1), lambda i: (0, 0)),
        ],
        out_specs=pl.BlockSpec((_S, _D), lambda i: (0, 0)),
        out_shape=jax.ShapeDtypeStruct((_S, _D), jnp.float32),
        scratch_shapes=[
            pltpu.VMEM((_S, _D), jnp.float32),
        ],
    )(x2_bf, batch, weight_matrix, sums, counts)
